# baseline (speedup 1.0000x reference)
import jax
import jax.numpy as jnp
from jax.experimental import pallas as pl
from jax.experimental.pallas import tpu as pltpu

_NUM_MODALITIES = 5
_EMBED_DIM = 1024


def _row_dma(idx_ref, emb_hbm, out_hbm, sem0, sem1):
    i = idx_ref[0]
    half = _EMBED_DIM // 2
    lo = pltpu.make_async_copy(emb_hbm.at[i, pl.ds(0, half)],
                               out_hbm.at[pl.ds(0, half)], sem0)
    hi = pltpu.make_async_copy(emb_hbm.at[i, pl.ds(half, half)],
                               out_hbm.at[pl.ds(half, half)], sem1)
    lo.start()
    hi.start()
    lo.wait()
    hi.wait()


def kernel(modality_embeddings, modality_idx):
    idx = jnp.asarray(modality_idx, dtype=jnp.int32).reshape((1,))
    return pl.pallas_call(
        _row_dma,
        in_specs=[
            pl.BlockSpec(memory_space=pltpu.SMEM),
            pl.BlockSpec(memory_space=pl.ANY),
        ],
        out_specs=pl.BlockSpec(memory_space=pl.ANY),
        out_shape=jax.ShapeDtypeStruct((_EMBED_DIM,),
                                       modality_embeddings.dtype),
        scratch_shapes=[pltpu.SemaphoreType.DMA, pltpu.SemaphoreType.DMA],
    )(idx, modality_embeddings)


# speedup vs baseline: 1.0116x; 1.0116x over previous
import jax
import jax.numpy as jnp
from jax.experimental import pallas as pl
from jax.experimental.pallas import tpu as pltpu

_NUM_MODALITIES = 5
_EMBED_DIM = 1024


def _row_dma(idx_ref, emb_hbm, out_hbm, sem):
    i = idx_ref[0]
    pltpu.make_async_copy(emb_hbm.at[i], out_hbm, sem).start()
    pltpu.make_async_copy(emb_hbm.at[i], out_hbm, sem).wait()


def kernel(modality_embeddings, modality_idx):
    idx = jnp.asarray(modality_idx, dtype=jnp.int32).reshape((1,))
    return pl.pallas_call(
        _row_dma,
        in_specs=[
            pl.BlockSpec(memory_space=pltpu.SMEM),
            pl.BlockSpec(memory_space=pl.ANY),
        ],
        out_specs=pl.BlockSpec(memory_space=pl.ANY),
        out_shape=jax.ShapeDtypeStruct((_EMBED_DIM,),
                                       modality_embeddings.dtype),
        scratch_shapes=[pltpu.SemaphoreType.DMA],
    )(idx, modality_embeddings)
